# baseline (speedup 1.0000x reference)
import functools

import jax
import jax.numpy as jnp
from jax import lax
from jax.experimental import pallas as pl
from jax.experimental.pallas import tpu as pltpu
from jax.experimental.pallas import tpu_sc as plsc

D = 64
L_SEQ = 200
B = 1024
NUM_CORES = 2
NUM_SUBCORES = 16
LANES = 16
QTOK = 256
NQ = B // QTOK


@functools.lru_cache(maxsize=None)
def _build_call():
    mesh = plsc.VectorSubcoreMesh(core_axis_name="c", subcore_axis_name="s")

    @functools.partial(
        pl.kernel,
        mesh=mesh,
        out_type=jax.ShapeDtypeStruct((L_SEQ, 8, 8, 8, 128), jnp.float32),
        compiler_params=pltpu.CompilerParams(needs_layout_passes=False),
        scratch_types=[
            pltpu.VMEM((8, 128), jnp.int32),
            pltpu.VMEM((128,), jnp.float32),
            pltpu.VMEM((QTOK // 2, 128), jnp.float32),
            pltpu.VMEM((QTOK // 2, 128), jnp.float32),
            pltpu.VMEM((8, 2, 8, 128), jnp.float32),
            pltpu.VMEM((8, 2, 8, 128), jnp.float32),
            pltpu.SemaphoreType.DMA,
            pltpu.SemaphoreType.DMA,
            pltpu.SemaphoreType.DMA,
        ],
    )
    def emb(ids5, tbl, pos2, out5, ids_s, pos_row,
            g0, g1, s0, s1, isem, gsem, osem):
        wid = lax.axis_index("s") * NUM_CORES + lax.axis_index("c")
        n_l = jnp.where(wid < 8, 7, 6)
        gbufs = (g0, g1)
        sbufs = (s0, s1)
        iota16 = lax.iota(jnp.int32, LANES)
        rvs = [bg * 8 + iota16 // 2 for bg in range(16)]
        par_static = (iota16 & 1) * 64

        def l_body(k, carry):
            l = wid + 32 * k
            tl = l // 8
            sl = l % 8

            pdesc = pltpu.async_copy(pos2.at[l // 2], pos_row, isem)
            idescs = [
                pltpu.async_copy(ids5.at[tl, tb, sl], ids_s.at[tb], isem)
                for tb in range(8)
            ]
            pdesc.wait()
            for dsc in idescs:
                dsc.wait()

            def fire_quarter(q):
                def f_body(i, c):
                    base = q * QTOK + i * LANES
                    vv = ids_s[base // 128, pl.ds(base % 128, LANES)]
                    for u in range(LANES):
                        v = vv[u]
                        pltpu.async_copy(
                            tbl.at[v],
                            gbufs[q % 2].at[i * 8 + u // 2,
                                            pl.ds((u % 2) * D, D)],
                            gsem,
                        )
                    return c

                lax.fori_loop(0, QTOK // LANES, f_body, 0)

            def drain_quarter(q):
                def d_body(i, c):
                    pltpu.make_async_copy(
                        tbl.at[0],
                        gbufs[q % 2].at[0, pl.ds(0, D)],
                        gsem,
                    ).wait()
                    return c

                lax.fori_loop(0, QTOK, d_body, 0)

            fire_quarter(0)
            sdescs = [None, None]
            for q in range(NQ):
                gbuf = gbufs[q % 2]
                sbuf = sbufs[q % 2]
                drain_quarter(q)
                if q + 1 < NQ:
                    fire_quarter(q + 1)
                if sdescs[q % 2] is not None:
                    sdescs[q % 2].wait()
                    sdescs[q % 2] = None

                poff = (l % 2) * D

                def d_body(dd, c):
                    ti = dd // 8
                    s2 = dd % 8
                    dsplat = jnp.full((LANES,), dd, jnp.int32)
                    ps = plsc.load_gather(
                        pos_row, [jnp.full((LANES,), poff + dd, jnp.int32)]
                    )
                    for bg in range(16):
                        cv = par_static + dsplat
                        g = plsc.load_gather(gbuf, [rvs[bg], cv])
                        sbuf[ti, bg // 8, s2, pl.ds((bg % 8) * LANES, LANES)] = (
                            g + ps
                        )
                    return c

                lax.fori_loop(0, D, d_body, 0)

                sdescs[q % 2] = pltpu.async_copy(
                    sbuf, out5.at[l, :, pl.ds(2 * q, 2)], osem
                )
            for dsc in sdescs:
                if dsc is not None:
                    dsc.wait()
            return carry

        lax.fori_loop(0, n_l, l_body, 0)

    return emb


def kernel(input_ids, token_table, position_table):
    ids5 = input_ids.astype(jnp.int32).reshape(8, 128, 25, 8).transpose(2, 0, 3, 1)
    pos2 = position_table.reshape(100, 128)
    out5 = _build_call()(ids5, token_table, pos2)
    return out5.transpose(2, 4, 0, 1, 3).reshape(B, L_SEQ, D)
